# baseline (speedup 1.0000x reference)
import functools

import jax
import jax.numpy as jnp
from jax import lax
from jax.experimental import pallas as pl
from jax.experimental.pallas import tpu as pltpu
from jax.experimental.pallas import tpu_sc as plsc

B = 16384
D = 32
CHUNK = 128
_info = plsc.get_sparse_core_info()
NC, NS = _info.num_cores, _info.num_subcores
NW = NC * NS
BPW = B // NW
KPW = BPW // CHUNK
NCHUNKS = B // CHUNK


@functools.partial(
    pl.kernel,
    mesh=plsc.VectorSubcoreMesh(core_axis_name="c", subcore_axis_name="s"),
    compiler_params=pltpu.CompilerParams(use_tc_tiling_on_sc=False),
    out_type=[
        jax.ShapeDtypeStruct((NCHUNKS, CHUNK, D), jnp.float32),
        jax.ShapeDtypeStruct((NCHUNKS, CHUNK, D), jnp.float32),
    ],
    scratch_types=[
        pltpu.VMEM((KPW, CHUNK), jnp.int32),
        pltpu.VMEM((KPW, CHUNK), jnp.int32),
        pltpu.VMEM((KPW, CHUNK, D), jnp.float32),
        pltpu.VMEM((KPW, CHUNK, D), jnp.float32),
        pltpu.SemaphoreType.DMA,
        pltpu.SemaphoreType.DMA,
    ],
)
def _sc_gather(uid_hbm, iid_hbm, utab_hbm, itab_hbm, uout_hbm, iout_hbm,
               uidx_v, iidx_v, urows_v, irows_v, sem_u, sem_i):
    wid = lax.axis_index("s") * NC + lax.axis_index("c")
    c0 = wid * KPW
    pltpu.sync_copy(uid_hbm.at[pl.ds(c0, KPW)], uidx_v)
    pltpu.sync_copy(iid_hbm.at[pl.ds(c0, KPW)], iidx_v)
    copies = []
    for k in range(KPW):
        copies.append(pltpu.async_copy(utab_hbm.at[uidx_v.at[k]],
                                       urows_v.at[k], sem_u))
        copies.append(pltpu.async_copy(itab_hbm.at[iidx_v.at[k]],
                                       irows_v.at[k], sem_i))
    for c in copies:
        c.wait()
    pltpu.sync_copy(urows_v, uout_hbm.at[pl.ds(c0, KPW)])
    pltpu.sync_copy(irows_v, iout_hbm.at[pl.ds(c0, KPW)])


BLK = 2048


def _mlp_body(xu_ref, xi_ref, w1u_ref, w1i_ref, b1_ref, w2_ref, b2_ref,
              w3_ref, b3_ref, o_ref):
    h = jnp.dot(xu_ref[...], w1u_ref[...], preferred_element_type=jnp.float32)
    h += jnp.dot(xi_ref[...], w1i_ref[...], preferred_element_type=jnp.float32)
    h = jnp.maximum(h + b1_ref[...], 0.0)
    h = jnp.dot(h, w2_ref[...], preferred_element_type=jnp.float32)
    h = jnp.maximum(h + b2_ref[...], 0.0)
    o = jnp.sum(h * w3_ref[...], axis=1, keepdims=True) + b3_ref[...]
    o_ref[...] = o


def _mlp(xu, xi, w1u, w1i, b1, w2, b2, w3t, b3):
    grid = (B // BLK,)
    return pl.pallas_call(
        _mlp_body,
        grid=grid,
        in_specs=[
            pl.BlockSpec((BLK, D), lambda i: (i, 0)),
            pl.BlockSpec((BLK, D), lambda i: (i, 0)),
            pl.BlockSpec((D, 256), lambda i: (0, 0)),
            pl.BlockSpec((D, 256), lambda i: (0, 0)),
            pl.BlockSpec((1, 256), lambda i: (0, 0)),
            pl.BlockSpec((256, 64), lambda i: (0, 0)),
            pl.BlockSpec((1, 64), lambda i: (0, 0)),
            pl.BlockSpec((1, 64), lambda i: (0, 0)),
            pl.BlockSpec((1, 1), lambda i: (0, 0)),
        ],
        out_specs=pl.BlockSpec((BLK, 1), lambda i: (i, 0)),
        out_shape=jax.ShapeDtypeStruct((B, 1), jnp.float32),
    )(xu, xi, w1u, w1i, b1, w2, b2, w3t, b3)


def kernel(user_id, item_id, user_table, item_table, W1, b1, W2, b2, W3, b3):
    uid = user_id.astype(jnp.int32).reshape(NCHUNKS, CHUNK)
    iid = item_id.astype(jnp.int32).reshape(NCHUNKS, CHUNK)
    uemb, iemb = _sc_gather(uid, iid, user_table, item_table)
    xu = uemb.reshape(B, D)
    xi = iemb.reshape(B, D)
    return _mlp(xu, xi, W1[:D, :], W1[D:, :], b1.reshape(1, 256),
                W2, b2.reshape(1, 64), W3.reshape(1, 64), b3.reshape(1, 1))
